# baseline (speedup 1.0000x reference)
import functools

import jax
import jax.numpy as jnp
from jax import lax
from jax.experimental import pallas as pl
from jax.experimental.pallas import tpu as pltpu
from jax.experimental.pallas import tpu_sc as plsc

_IDX_CHUNK = 128


def _sc_gather(idx3d, emb_table):
    num_workers, chunks_per_w, chunk = idx3d.shape
    _, emb = emb_table.shape

    mesh = plsc.VectorSubcoreMesh(core_axis_name="c", subcore_axis_name="s")

    @functools.partial(
        pl.kernel,
        out_type=jax.ShapeDtypeStruct((num_workers * chunks_per_w, chunk, emb), jnp.float32),
        mesh=mesh,
        scratch_types=[
            pltpu.VMEM((chunks_per_w, chunk), jnp.int32),
            pltpu.VMEM((chunks_per_w, chunk, emb), jnp.float32),
            pltpu.SemaphoreType.DMA,
        ],
        compiler_params=pltpu.CompilerParams(use_tc_tiling_on_sc=False),
    )
    def gather_kernel(idx_hbm, table_hbm, out_hbm, idx_v, rows_v, sem):
        num_cores = jax.lax.axis_size("c")
        wid = lax.axis_index("s") * num_cores + lax.axis_index("c")
        pltpu.sync_copy(idx_hbm.at[wid], idx_v)
        copies = [
            pltpu.async_copy(table_hbm.at[idx_v.at[j]], rows_v.at[j], sem)
            for j in range(chunks_per_w)
        ]
        for c in copies:
            c.wait()
        pltpu.sync_copy(rows_v, out_hbm.at[pl.ds(wid * chunks_per_w, chunks_per_w)])

    return gather_kernel(idx3d, emb_table)


def _projection(flat, dense_w, dense_b, block_n, n_out_bufs=3):
    m, k = flat.shape
    n = dense_w.shape[1]
    nsteps = n // block_n
    pad = (nsteps + 1) * block_n - n
    bias_rows = jnp.pad(dense_b, (0, pad)).reshape(nsteps + 1, 1, block_n)

    def mm_kernel(flat_ref, w_hbm, b_ref, out_hbm, w_buf, o_buf, w_sem, o_sem):
        i = pl.program_id(0)

        def w_copy(step, slot):
            return pltpu.make_async_copy(
                w_hbm.at[:, pl.ds(step * block_n, block_n)],
                w_buf.at[slot],
                w_sem.at[slot],
            )

        def o_copy(step, slot):
            return pltpu.make_async_copy(
                o_buf.at[slot],
                out_hbm.at[:, pl.ds(step * block_n, block_n)],
                o_sem.at[slot],
            )

        @pl.when(i == 0)
        def _():
            w_copy(0, 0).start(priority=0)

        for s in range(2):
            @pl.when((i + 1 < nsteps) & ((i + 1) % 2 == s))
            def _(s=s):
                w_copy(i + 1, s).start(priority=s)

        slot_w = i % 2
        slot_o = i % n_out_bufs

        @pl.when(i < nsteps)
        def _():
            @pl.when(i >= n_out_bufs)
            def _():
                o_copy(i - n_out_bufs, slot_o).wait()

            w_copy(i, slot_w).wait()
            o_buf[slot_o] = (
                jnp.dot(
                    flat_ref[...], w_buf[slot_w], preferred_element_type=jnp.float32
                )
                + b_ref[0]
            )
            for s in range(n_out_bufs):
                @pl.when(slot_o == s)
                def _(s=s):
                    o_copy(i, s).start(priority=s % 2)

        @pl.when(i == nsteps)
        def _():
            for d in range(n_out_bufs):
                step = nsteps - n_out_bufs + d
                o_copy(step, step % n_out_bufs).wait()

            def tail_body(w_ref, out_ref):
                out_ref[...] = (
                    jnp.dot(
                        flat_ref[...], w_ref[...], preferred_element_type=jnp.float32
                    )
                    + b_ref[0]
                )

            pltpu.emit_pipeline(
                tail_body,
                grid=(1,),
                in_specs=[pl.BlockSpec((k, block_n), lambda j: (0, nsteps))],
                out_specs=[pl.BlockSpec((m, block_n), lambda j: (0, nsteps))],
            )(w_hbm, out_hbm)

    return pl.pallas_call(
        mm_kernel,
        grid=(nsteps + 1,),
        in_specs=[
            pl.BlockSpec((m, k), lambda i: (0, 0)),
            pl.BlockSpec(memory_space=pl.ANY),
            pl.BlockSpec((1, 1, block_n), lambda i: (i, 0, 0)),
        ],
        out_specs=pl.BlockSpec(memory_space=pl.ANY),
        out_shape=jax.ShapeDtypeStruct((m, n), jnp.float32),
        scratch_shapes=[
            pltpu.VMEM((2, k, block_n), jnp.float32),
            pltpu.VMEM((n_out_bufs, m, block_n), jnp.float32),
            pltpu.SemaphoreType.DMA((2,)),
            pltpu.SemaphoreType.DMA((n_out_bufs,)),
        ],
        compiler_params=pltpu.CompilerParams(
            dimension_semantics=("arbitrary",),
        ),
    )(flat, dense_w, bias_rows)


def kernel(context, emb_table, dense_w, dense_b):
    batch, ctx_len = context.shape
    vocab, emb = emb_table.shape
    total = batch * ctx_len
    info = plsc.get_sparse_core_info()
    num_workers = info.num_cores * info.num_subcores
    idx3d = context.astype(jnp.int32).reshape(
        num_workers, total // (num_workers * _IDX_CHUNK), _IDX_CHUNK
    )
    rows = _sc_gather(idx3d, emb_table)
    flat = rows.reshape(batch, ctx_len * emb)
    logits = _projection(flat, dense_w, dense_b, block_n=2048)
    return logits


# speedup vs baseline: 1.8681x; 1.8681x over previous
import functools

import jax
import jax.numpy as jnp
from jax import lax
from jax.experimental import pallas as pl
from jax.experimental.pallas import tpu as pltpu
from jax.experimental.pallas import tpu_sc as plsc

_IDX_CHUNK = 128


def _sc_gather(idx3d, emb_table):
    num_workers, chunks_per_w, chunk = idx3d.shape
    _, emb = emb_table.shape

    mesh = plsc.VectorSubcoreMesh(core_axis_name="c", subcore_axis_name="s")

    @functools.partial(
        pl.kernel,
        out_type=jax.ShapeDtypeStruct((num_workers * chunks_per_w, chunk, emb), jnp.float32),
        mesh=mesh,
        scratch_types=[
            pltpu.VMEM((chunks_per_w, chunk), jnp.int32),
            pltpu.VMEM((chunks_per_w, chunk, emb), jnp.float32),
            pltpu.SemaphoreType.DMA,
        ],
        compiler_params=pltpu.CompilerParams(use_tc_tiling_on_sc=False),
    )
    def gather_kernel(idx_hbm, table_hbm, out_hbm, idx_v, rows_v, sem):
        num_cores = jax.lax.axis_size("c")
        wid = lax.axis_index("s") * num_cores + lax.axis_index("c")
        pltpu.sync_copy(idx_hbm.at[wid], idx_v)
        copies = [
            pltpu.async_copy(table_hbm.at[idx_v.at[j]], rows_v.at[j], sem)
            for j in range(chunks_per_w)
        ]
        for c in copies:
            c.wait()
        pltpu.sync_copy(rows_v, out_hbm.at[pl.ds(wid * chunks_per_w, chunks_per_w)])

    return gather_kernel(idx3d, emb_table)


def _projection_t(flat_t, dense_w, dense_b, block_n, n_out_bufs=3):
    k, m = flat_t.shape
    n = dense_w.shape[1]
    nsteps = n // block_n
    n_tail = n - nsteps * block_n
    w_tail = lax.slice(dense_w, (0, nsteps * block_n), (k, n))
    pad = (nsteps + 1) * block_n - n
    bias_col = jnp.pad(dense_b, (0, pad)).reshape(nsteps + 1, block_n, 1)

    def mm_kernel(flat_ref, wt_ref, b_ref, w_hbm, out_hbm, w_buf, o_buf, w_sem, o_sem):
        i = pl.program_id(0)

        def w_copy(step, slot):
            return pltpu.make_async_copy(
                w_hbm.at[:, pl.ds(step * block_n, block_n)],
                w_buf.at[slot],
                w_sem.at[slot],
            )

        def o_copy(step, slot):
            return pltpu.make_async_copy(
                o_buf.at[slot],
                out_hbm.at[pl.ds(step * block_n, block_n), :],
                o_sem.at[slot],
            )

        def o_copy_tail(slot):
            return pltpu.make_async_copy(
                o_buf.at[slot, pl.ds(0, n_tail)],
                out_hbm.at[pl.ds(nsteps * block_n, n_tail), :],
                o_sem.at[slot],
            )

        @pl.when(i == 0)
        def _():
            w_copy(0, 0).start(priority=0)

        for s in range(2):
            @pl.when((i + 1 < nsteps) & ((i + 1) % 2 == s))
            def _(s=s):
                w_copy(i + 1, s).start(priority=s)

        slot_w = i % 2
        slot_o = i % n_out_bufs

        @pl.when(i >= n_out_bufs)
        def _():
            o_copy(i - n_out_bufs, slot_o).wait()

        @pl.when(i < nsteps)
        def _():
            w_copy(i, slot_w).wait()
            o_buf[slot_o] = (
                lax.dot_general(
                    w_buf[slot_w],
                    flat_ref[...],
                    (((0,), (0,)), ((), ())),
                    preferred_element_type=jnp.float32,
                )
                + b_ref[0]
            )
            for s in range(n_out_bufs):
                @pl.when(slot_o == s)
                def _(s=s):
                    o_copy(i, s).start(priority=s % 2)

        @pl.when(i == nsteps)
        def _():
            o_buf[slot_o, pl.ds(0, n_tail)] = (
                lax.dot_general(
                    wt_ref[...],
                    flat_ref[...],
                    (((0,), (0,)), ((), ())),
                    preferred_element_type=jnp.float32,
                )
                + b_ref[0, pl.ds(0, n_tail)]
            )
            for s in range(n_out_bufs):
                @pl.when(slot_o == s)
                def _(s=s):
                    o_copy_tail(s).start(priority=s % 2)

            for d in range(n_out_bufs - 1):
                step = nsteps - (n_out_bufs - 1) + d
                o_copy(step, step % n_out_bufs).wait()
            for s in range(n_out_bufs):
                @pl.when(slot_o == s)
                def _(s=s):
                    o_copy_tail(s).wait()

    return pl.pallas_call(
        mm_kernel,
        grid=(nsteps + 1,),
        in_specs=[
            pl.BlockSpec((k, m), lambda i: (0, 0)),
            pl.BlockSpec((k, n_tail), lambda i: (0, 0)),
            pl.BlockSpec((1, block_n, 1), lambda i: (i, 0, 0)),
            pl.BlockSpec(memory_space=pl.ANY),
        ],
        out_specs=pl.BlockSpec(memory_space=pl.ANY),
        out_shape=jax.ShapeDtypeStruct((n, m), jnp.float32),
        scratch_shapes=[
            pltpu.VMEM((2, k, block_n), jnp.float32),
            pltpu.VMEM((n_out_bufs, block_n, m), jnp.float32),
            pltpu.SemaphoreType.DMA((2,)),
            pltpu.SemaphoreType.DMA((n_out_bufs,)),
        ],
        compiler_params=pltpu.CompilerParams(
            dimension_semantics=("arbitrary",),
        ),
    )(flat_t, w_tail, bias_col, dense_w)


def kernel(context, emb_table, dense_w, dense_b):
    batch, ctx_len = context.shape
    vocab, emb = emb_table.shape
    total = batch * ctx_len
    info = plsc.get_sparse_core_info()
    num_workers = info.num_cores * info.num_subcores
    idx3d = context.astype(jnp.int32).reshape(
        num_workers, total // (num_workers * _IDX_CHUNK), _IDX_CHUNK
    )
    rows = _sc_gather(idx3d, emb_table)
    flat_t = rows.reshape(batch, ctx_len * emb).T
    logits_t = _projection_t(flat_t, dense_w, dense_b, block_n=2048)
    return logits_t.T


# speedup vs baseline: 1.8757x; 1.0041x over previous
import functools

import jax
import jax.numpy as jnp
from jax import lax
from jax.experimental import pallas as pl
from jax.experimental.pallas import tpu as pltpu
from jax.experimental.pallas import tpu_sc as plsc

_IDX_CHUNK = 128


def _sc_gather(idx3d, emb_table):
    num_workers, chunks_per_w, chunk = idx3d.shape
    _, emb = emb_table.shape

    mesh = plsc.VectorSubcoreMesh(core_axis_name="c", subcore_axis_name="s")

    @functools.partial(
        pl.kernel,
        out_type=jax.ShapeDtypeStruct(
            (num_workers * chunks_per_w, chunk, emb), jnp.float32
        ),
        mesh=mesh,
        scratch_types=[
            pltpu.VMEM((chunks_per_w, chunk), jnp.int32),
            pltpu.VMEM((chunks_per_w, chunk, emb), jnp.float32),
            pltpu.SemaphoreType.DMA,
        ],
        compiler_params=pltpu.CompilerParams(use_tc_tiling_on_sc=False),
    )
    def gather_kernel(idx_hbm, table_hbm, out_hbm, idx_v, rows_v, sem):
        num_cores = jax.lax.axis_size("c")
        wid = lax.axis_index("s") * num_cores + lax.axis_index("c")
        pltpu.sync_copy(idx_hbm.at[wid], idx_v)
        copies = [
            pltpu.async_copy(table_hbm.at[idx_v.at[j]], rows_v.at[j], sem)
            for j in range(chunks_per_w)
        ]
        for c in copies:
            c.wait()
        pltpu.sync_copy(rows_v, out_hbm.at[pl.ds(wid * chunks_per_w, chunks_per_w)])

    return gather_kernel(idx3d, emb_table)


def _projection_t(flat, dense_w, dense_b, block_n, n_out_bufs=3):
    m, k = flat.shape
    n = dense_w.shape[1]
    nsteps = n // block_n
    n_tail = n - nsteps * block_n
    w_tail = lax.slice(dense_w, (0, nsteps * block_n), (k, n))
    pad = (nsteps + 1) * block_n - n
    bias_col = jnp.pad(dense_b, (0, pad)).reshape(nsteps + 1, block_n, 1)

    def mm_kernel(flat_ref, wt_ref, b_ref, w_hbm, out_hbm, w_buf, o_buf, w_sem, o_sem):
        i = pl.program_id(0)

        def w_copy(step, slot):
            return pltpu.make_async_copy(
                w_hbm.at[:, pl.ds(step * block_n, block_n)],
                w_buf.at[slot],
                w_sem.at[slot],
            )

        def o_copy(step, slot):
            return pltpu.make_async_copy(
                o_buf.at[slot],
                out_hbm.at[pl.ds(step * block_n, block_n), :],
                o_sem.at[slot],
            )

        def o_copy_tail(slot):
            return pltpu.make_async_copy(
                o_buf.at[slot, pl.ds(0, n_tail)],
                out_hbm.at[pl.ds(nsteps * block_n, n_tail), :],
                o_sem.at[slot],
            )

        @pl.when(i == 0)
        def _():
            w_copy(0, 0).start(priority=0)

        for s in range(2):
            @pl.when((i + 1 < nsteps) & ((i + 1) % 2 == s))
            def _(s=s):
                w_copy(i + 1, s).start(priority=s)

        slot_w = i % 2
        slot_o = i % n_out_bufs

        @pl.when(i >= n_out_bufs)
        def _():
            o_copy(i - n_out_bufs, slot_o).wait()

        @pl.when(i < nsteps)
        def _():
            w_copy(i, slot_w).wait()
            o_buf[slot_o] = (
                lax.dot_general(
                    w_buf[slot_w],
                    flat_ref[...],
                    (((0,), (1,)), ((), ())),
                    preferred_element_type=jnp.float32,
                )
                + b_ref[0]
            )
            for s in range(n_out_bufs):
                @pl.when(slot_o == s)
                def _(s=s):
                    o_copy(i, s).start(priority=s % 2)

        @pl.when(i == nsteps)
        def _():
            o_buf[slot_o, pl.ds(0, n_tail)] = (
                lax.dot_general(
                    wt_ref[...],
                    flat_ref[...],
                    (((0,), (1,)), ((), ())),
                    preferred_element_type=jnp.float32,
                )
                + b_ref[0, pl.ds(0, n_tail)]
            )
            for s in range(n_out_bufs):
                @pl.when(slot_o == s)
                def _(s=s):
                    o_copy_tail(s).start(priority=s % 2)

            for d in range(n_out_bufs - 1):
                step = nsteps - (n_out_bufs - 1) + d
                o_copy(step, step % n_out_bufs).wait()
            for s in range(n_out_bufs):
                @pl.when(slot_o == s)
                def _(s=s):
                    o_copy_tail(s).wait()

    return pl.pallas_call(
        mm_kernel,
        grid=(nsteps + 1,),
        in_specs=[
            pl.BlockSpec((m, k), lambda i: (0, 0)),
            pl.BlockSpec((k, n_tail), lambda i: (0, 0)),
            pl.BlockSpec((1, block_n, 1), lambda i: (i, 0, 0)),
            pl.BlockSpec(memory_space=pl.ANY),
        ],
        out_specs=pl.BlockSpec(memory_space=pl.ANY),
        out_shape=jax.ShapeDtypeStruct((n, m), jnp.float32),
        scratch_shapes=[
            pltpu.VMEM((2, k, block_n), jnp.float32),
            pltpu.VMEM((n_out_bufs, block_n, m), jnp.float32),
            pltpu.SemaphoreType.DMA((2,)),
            pltpu.SemaphoreType.DMA((n_out_bufs,)),
        ],
        compiler_params=pltpu.CompilerParams(
            dimension_semantics=("arbitrary",),
        ),
    )(flat, w_tail, bias_col, dense_w)


def kernel(context, emb_table, dense_w, dense_b):
    batch, ctx_len = context.shape
    vocab, emb = emb_table.shape
    total = batch * ctx_len
    info = plsc.get_sparse_core_info()
    num_workers = info.num_cores * info.num_subcores
    idx3d = context.astype(jnp.int32).reshape(
        num_workers, total // (num_workers * _IDX_CHUNK), _IDX_CHUNK
    )
    rows = _sc_gather(idx3d, emb_table)
    flat = rows.reshape(batch, ctx_len * emb)
    logits_t = _projection_t(flat, dense_w, dense_b, block_n=2048)
    return logits_t.T
